# baseline (speedup 1.0000x reference)
import functools

import jax
import jax.numpy as jnp
from jax import lax
from jax.experimental import pallas as pl
from jax.experimental.pallas import tpu as pltpu
from jax.experimental.pallas import tpu_sc as plsc

_D = 1000
_B, _T = 1024, 50
_NC, _NS = 2, 16
_NW = _NC * _NS
_CHUNKS = 4


def _sc_gather(x, prob):
  nb = x.shape[0]
  bpw = nb // _NW
  mesh = plsc.VectorSubcoreMesh(core_axis_name="c", subcore_axis_name="s")

  @functools.partial(
      pl.kernel,
      out_type=jax.ShapeDtypeStruct((nb, _T, _D), jnp.float32),
      mesh=mesh,
      scratch_types=[
          pltpu.VMEM((bpw, _T), jnp.int32),
          pltpu.VMEM((_T, _D), jnp.float32),
          pltpu.VMEM((_T, _D), jnp.float32),
          pltpu.SemaphoreType.DMA,
          pltpu.SemaphoreType.DMA,
          pltpu.SemaphoreType.DMA,
          pltpu.SemaphoreType.DMA,
      ],
      compiler_params=pltpu.CompilerParams(use_tc_tiling_on_sc=False),
  )
  def body(idx_hbm, table_hbm, out_hbm, idx_v, rows0, rows1, g0, g1, s0, s1):
    wid = lax.axis_index("s") * _NC + lax.axis_index("c")
    b0 = wid * bpw
    pltpu.sync_copy(idx_hbm.at[pl.ds(b0, bpw)], idx_v)

    bufs = (rows0, rows1)
    gsems = (g0, g1)
    ssems = (s0, s1)

    def gather(c, p):
      return pltpu.make_async_copy(
          table_hbm.at[idx_v.at[c]], bufs[p], gsems[p])

    def scatter(c, p):
      return pltpu.make_async_copy(bufs[p], out_hbm.at[b0 + c], ssems[p])

    gather(0, 0).start()
    gather(1, 1).start()

    def step(jj, carry):
      c0 = 2 * jj
      gather(c0, 0).wait()
      scatter(c0, 0).start()
      gather(c0 + 1, 1).wait()
      scatter(c0 + 1, 1).start()
      scatter(c0, 0).wait()
      gather(c0 + 2, 0).start()
      scatter(c0 + 1, 1).wait()
      gather(c0 + 3, 1).start()
      return carry

    lax.fori_loop(0, bpw // 2 - 1, step, 0)

    cl = bpw - 2
    gather(cl, 0).wait()
    scatter(cl, 0).start()
    gather(cl + 1, 1).wait()
    scatter(cl + 1, 1).start()
    scatter(cl, 0).wait()
    scatter(cl + 1, 1).wait()

  return body(x, prob)


def kernel(x, prob):
  step = _B // _CHUNKS
  out = jnp.zeros((_B, _T, _D), jnp.float32)
  for i in range(_CHUNKS):
    part = _sc_gather(x[i * step:(i + 1) * step], prob)
    out = lax.dynamic_update_slice(out, part, (i * step, 0, 0))
  return out


# speedup vs baseline: 2.0578x; 2.0578x over previous
import functools

import jax
import jax.numpy as jnp
from jax import lax
from jax.experimental import pallas as pl
from jax.experimental.pallas import tpu as pltpu
from jax.experimental.pallas import tpu_sc as plsc

_D = 1000
_DA = 896
_DB = 128
_B, _T = 1024, 50
_NC, _NS = 2, 16
_NW = _NC * _NS


def _sc_gather(x, table_a, table_b):
  nb = x.shape[0]
  bpw = nb // _NW
  mesh = plsc.VectorSubcoreMesh(core_axis_name="c", subcore_axis_name="s")

  @functools.partial(
      pl.kernel,
      out_type=jax.ShapeDtypeStruct((nb, _T, _D), jnp.float32),
      mesh=mesh,
      scratch_types=[
          pltpu.VMEM((bpw, _T), jnp.int32),
          pltpu.VMEM((_T, _DA), jnp.float32),
          pltpu.VMEM((_T, _DA), jnp.float32),
          pltpu.VMEM((_T, _DB), jnp.float32),
          pltpu.VMEM((_T, _DB), jnp.float32),
          pltpu.SemaphoreType.DMA,
          pltpu.SemaphoreType.DMA,
          pltpu.SemaphoreType.DMA,
          pltpu.SemaphoreType.DMA,
      ],
  )
  def body(idx_hbm, ta_hbm, tb_hbm, out_hbm, idx_v,
           a0, a1, b0_, b1_, g0, g1, s0, s1):
    wid = lax.axis_index("s") * _NC + lax.axis_index("c")
    b0 = wid * bpw
    pltpu.sync_copy(idx_hbm.at[pl.ds(b0, bpw)], idx_v)

    abufs = (a0, a1)
    bbufs = (b0_, b1_)
    gsems = (g0, g1)
    ssems = (s0, s1)

    class _Gather:
      def __init__(self, c, p):
        self.cps = [
            pltpu.make_async_copy(
                ta_hbm.at[idx_v.at[c]], abufs[p], gsems[p]),
            pltpu.make_async_copy(
                tb_hbm.at[idx_v.at[c]], bbufs[p], gsems[p]),
        ]

      def start(self):
        for cp in self.cps:
          cp.start()

      def wait(self):
        for cp in self.cps:
          cp.wait()

    class _Scatter:
      def __init__(self, c, p):
        tail = _D - _DA
        cps = [
            pltpu.make_async_copy(
                abufs[p].at[pl.ds(0, 48), :],
                out_hbm.at[b0 + c, pl.ds(0, 48), pl.ds(0, _DA)],
                ssems[p]),
            pltpu.make_async_copy(
                abufs[p].at[pl.ds(48, 2), :],
                out_hbm.at[b0 + c, pl.ds(48, 2), pl.ds(0, _DA)],
                ssems[p]),
        ]
        for t in range(_T):
          cps.append(pltpu.make_async_copy(
              bbufs[p].at[t, pl.ds(0, tail)],
              out_hbm.at[b0 + c, t, pl.ds(_DA, tail)],
              ssems[p]))
        self.cps = cps

      def start(self):
        for cp in self.cps:
          cp.start()

      def wait(self):
        for cp in self.cps:
          cp.wait()

    gather = _Gather
    scatter = _Scatter

    gather(0, 0).start()
    gather(1, 1).start()

    def step(jj, carry):
      c0 = 2 * jj
      gather(c0, 0).wait()
      scatter(c0, 0).start()
      gather(c0 + 1, 1).wait()
      scatter(c0 + 1, 1).start()
      scatter(c0, 0).wait()
      gather(c0 + 2, 0).start()
      scatter(c0 + 1, 1).wait()
      gather(c0 + 3, 1).start()
      return carry

    lax.fori_loop(0, bpw // 2 - 1, step, 0)

    cl = bpw - 2
    gather(cl, 0).wait()
    scatter(cl, 0).start()
    gather(cl + 1, 1).wait()
    scatter(cl + 1, 1).start()
    scatter(cl, 0).wait()
    scatter(cl + 1, 1).wait()

  return body(x, table_a, table_b)


def kernel(x, prob):
  table_a = prob[:, :_DA]
  table_b = jnp.pad(prob[:, _DA:], ((0, 0), (0, _DB - (_D - _DA))))
  return _sc_gather(x, table_a, table_b)


# speedup vs baseline: 2.1036x; 1.0223x over previous
import functools

import jax
import jax.numpy as jnp
from jax import lax
from jax.experimental import pallas as pl
from jax.experimental.pallas import tpu as pltpu
from jax.experimental.pallas import tpu_sc as plsc

_D = 1000
_DA = 896
_DB = 128
_B, _T = 1024, 50
_NC, _NS = 2, 16
_NW = _NC * _NS


def _sc_gather(x, table_a, table_b):
  nb = x.shape[0]
  bpw = nb // _NW
  mesh = plsc.VectorSubcoreMesh(core_axis_name="c", subcore_axis_name="s")

  @functools.partial(
      pl.kernel,
      out_type=jax.ShapeDtypeStruct((nb, _T, _D), jnp.float32),
      mesh=mesh,
      scratch_types=[
          pltpu.VMEM((bpw, _T), jnp.int32),
          pltpu.VMEM((_T, _DA), jnp.float32),
          pltpu.VMEM((_T, _DA), jnp.float32),
          pltpu.VMEM((_T, _DB), jnp.float32),
          pltpu.VMEM((_T, _DB), jnp.float32),
          pltpu.SemaphoreType.DMA,
          pltpu.SemaphoreType.DMA,
          pltpu.SemaphoreType.DMA,
          pltpu.SemaphoreType.DMA,
      ],
  )
  def body(idx_hbm, ta_hbm, tb_hbm, out_hbm, idx_v,
           a0, a1, b0_, b1_, g0, g1, s0, s1):
    wid = lax.axis_index("s") * _NC + lax.axis_index("c")
    b0 = wid * bpw
    pltpu.sync_copy(idx_hbm.at[pl.ds(b0, bpw)], idx_v)

    abufs = (a0, a1)
    bbufs = (b0_, b1_)
    gsems = (g0, g1)
    ssems = (s0, s1)

    class _Gather:
      def __init__(self, c, p):
        self.cps = [
            pltpu.make_async_copy(
                ta_hbm.at[idx_v.at[c]], abufs[p], gsems[p]),
            pltpu.make_async_copy(
                tb_hbm.at[idx_v.at[c]], bbufs[p], gsems[p]),
        ]

      def start(self):
        for cp in self.cps:
          cp.start()

      def wait(self):
        for cp in self.cps:
          cp.wait()

    class _Scatter:
      def __init__(self, c, p):
        tail = _D - _DA
        cps = [
            pltpu.make_async_copy(
                abufs[p].at[pl.ds(0, 48), :],
                out_hbm.at[b0 + c, pl.ds(0, 48), pl.ds(0, _DA)],
                ssems[p]),
        ]
        for k in range(2):
          for tc in range(7):
            cps.append(pltpu.make_async_copy(
                abufs[p].at[48 + k, pl.ds(128 * tc, 128)],
                out_hbm.at[b0 + c, 48 + k, pl.ds(128 * tc, 128)],
                ssems[p]))
        for t in range(_T):
          cps.append(pltpu.make_async_copy(
              bbufs[p].at[t, pl.ds(0, tail)],
              out_hbm.at[b0 + c, t, pl.ds(_DA, tail)],
              ssems[p]))
        self.cps = cps

      def start(self):
        for cp in self.cps:
          cp.start()

      def wait(self):
        for cp in self.cps:
          cp.wait()

    gather = _Gather
    scatter = _Scatter

    gather(0, 0).start()
    gather(1, 1).start()

    def step(jj, carry):
      c0 = 2 * jj
      gather(c0, 0).wait()
      scatter(c0, 0).start()
      gather(c0 + 1, 1).wait()
      scatter(c0 + 1, 1).start()
      scatter(c0, 0).wait()
      gather(c0 + 2, 0).start()
      scatter(c0 + 1, 1).wait()
      gather(c0 + 3, 1).start()
      return carry

    lax.fori_loop(0, bpw // 2 - 1, step, 0)

    cl = bpw - 2
    gather(cl, 0).wait()
    scatter(cl, 0).start()
    gather(cl + 1, 1).wait()
    scatter(cl + 1, 1).start()
    scatter(cl, 0).wait()
    scatter(cl + 1, 1).wait()

  return body(x, table_a, table_b)


def kernel(x, prob):
  table_a = prob[:, :_DA]
  table_b = jnp.pad(prob[:, _DA:], ((0, 0), (0, _DB - (_D - _DA))))
  return _sc_gather(x, table_a, table_b)
